# baseline (speedup 1.0000x reference)
import functools

import jax
import jax.numpy as jnp
from jax import lax
from jax.experimental import pallas as pl
from jax.experimental.pallas import tpu as pltpu
from jax.experimental.pallas import tpu_sc as plsc

BATCH = 16
SEQ_LEN = 2048
NUM_LAYERS = 3
DIM = 64
NUM_POS = BATCH * SEQ_LEN
VOCAB = 100000

_info = plsc.get_sparse_core_info()
_NC, _NS = _info.num_cores, _info.num_subcores
NW = _NC * _NS
POS_PER_W = NUM_POS // NW
CHUNK = 128
NCHUNK = POS_PER_W // CHUNK

_mesh = plsc.VectorSubcoreMesh(core_axis_name="c", subcore_axis_name="s")


HALF_VOCAB = VOCAB // 2


def _repack(w):
    rows = 1000
    nblk = HALF_VOCAB // rows

    def body(a_ref, b_ref, out_ref):
        out_ref[:, :DIM] = a_ref[...]
        out_ref[:, DIM:] = b_ref[...]

    return pl.pallas_call(
        body,
        grid=(nblk,),
        in_specs=[
            pl.BlockSpec((rows, DIM), lambda i: (i, 0)),
            pl.BlockSpec((rows, DIM), lambda i: (i + nblk, 0)),
        ],
        out_specs=pl.BlockSpec((rows, 2 * DIM), lambda i: (i, 0)),
        out_shape=jax.ShapeDtypeStruct((HALF_VOCAB, 2 * DIM), jnp.float32),
    )(w, w)


@functools.partial(
    pl.kernel,
    mesh=_mesh,
    out_type=jax.ShapeDtypeStruct((NUM_POS, NUM_LAYERS * DIM), jnp.float32),
    scratch_types=[
        pltpu.VMEM((NUM_LAYERS * POS_PER_W,), jnp.int32),
        pltpu.VMEM((NUM_LAYERS * POS_PER_W,), jnp.int32),
        pltpu.VMEM((CHUNK, 2 * DIM), jnp.float32),
        pltpu.VMEM((CHUNK, 2 * DIM), jnp.float32),
        pltpu.VMEM((CHUNK, NUM_LAYERS * DIM), jnp.float32),
        pltpu.VMEM((CHUNK, NUM_LAYERS * DIM), jnp.float32),
        pltpu.SemaphoreType.DMA,
        pltpu.SemaphoreType.DMA,
        pltpu.SemaphoreType.DMA,
        pltpu.SemaphoreType.DMA,
    ],
)
def _emb_lookup(
    tok_hbm, w6, w9, w12, out_hbm,
    tok_v, ixp_v, g0, g1, s0, s1, gsem0, gsem1, wsem0, wsem1,
):
    wid = lax.axis_index("c") * _NS + lax.axis_index("s")
    base = wid * POS_PER_W
    tables = (w6, w9, w12)
    gbufs = (g0, g1)
    gsems = (gsem0, gsem1)
    stages = (s0, s1)
    wsems = (wsem0, wsem1)

    for i in range(NUM_LAYERS):
        pltpu.sync_copy(
            tok_hbm.at[pl.ds(i * NUM_POS + base, POS_PER_W)],
            tok_v.at[pl.ds(i * POS_PER_W, POS_PER_W)],
        )

    def shift(j, cc):
        t = tok_v[pl.ds(j * 16, 16)]
        hi = ((t - HALF_VOCAB) >> 31) + 1
        ixp_v[pl.ds(j * 16, 16)] = t - hi * HALF_VOCAB
        return cc

    lax.fori_loop(0, NUM_LAYERS * POS_PER_W // 16, shift, 0)

    def issue_gather(i, c, buf, sem):
        off = i * POS_PER_W + c * CHUNK
        pltpu.async_copy(tables[i].at[ixp_v.at[pl.ds(off, CHUNK)]], buf, sem)

    def wait_gather(buf, sem):
        pltpu.make_async_copy(tables[0].at[pl.ds(0, CHUNK)], buf, sem).wait()

    def select(i, c, buf, st):
        def body(j, cc):
            tv = tok_v[pl.ds(i * POS_PER_W + c * CHUNK + j * 16, 16)]
            pv = (((tv - HALF_VOCAB) >> 31) + 1) * DIM
            for lane in range(16):
                off = pv[lane]
                p = j * 16 + lane
                for kk in range(DIM // 16):
                    st[p, pl.ds(i * DIM + kk * 16, 16)] = buf[
                        p, pl.ds(off + kk * 16, 16)
                    ]
            return cc

        lax.fori_loop(0, CHUNK // 16, body, 0)

    issue_gather(0, 0, gbufs[0], gsems[0])

    def chunk_pair(t, carry):
        for half in range(2):
            c = 2 * t + half
            st = stages[half]

            @pl.when(t > 0)
            def _():
                pltpu.make_async_copy(
                    st, out_hbm.at[pl.ds(base, CHUNK)], wsems[half]
                ).wait()

            for i in range(NUM_LAYERS):
                kpar = (3 * half + i) % 2
                cur, csem = gbufs[kpar], gsems[kpar]
                nxt, nsem = gbufs[1 - kpar], gsems[1 - kpar]
                if i < 2:
                    issue_gather(i + 1, c, nxt, nsem)
                elif half == 0:
                    issue_gather(0, c + 1, nxt, nsem)
                else:

                    @pl.when(t < NCHUNK // 2 - 1)
                    def _():
                        issue_gather(0, c + 1, nxt, nsem)

                wait_gather(cur, csem)
                select(i, c, cur, st)

            pltpu.async_copy(
                st, out_hbm.at[pl.ds(base + c * CHUNK, CHUNK)], wsems[half]
            )
        return carry

    lax.fori_loop(0, NCHUNK // 2, chunk_pair, 0)

    for half in range(2):
        pltpu.make_async_copy(
            stages[half], out_hbm.at[pl.ds(base, CHUNK)], wsems[half]
        ).wait()


def kernel(tokens, W6, W9, W12):
    tok_t = jnp.transpose(
        tokens.reshape(NUM_POS, NUM_LAYERS).astype(jnp.int32)
    ).reshape(NUM_LAYERS * NUM_POS)
    out = _emb_lookup(tok_t, _repack(W6), _repack(W9), _repack(W12))
    return out.reshape(BATCH, SEQ_LEN, NUM_LAYERS, DIM)


# speedup vs baseline: 1.2482x; 1.2482x over previous
import functools

import jax
import jax.numpy as jnp
from jax import lax
from jax.experimental import pallas as pl
from jax.experimental.pallas import tpu as pltpu
from jax.experimental.pallas import tpu_sc as plsc

BATCH = 16
SEQ_LEN = 2048
NUM_LAYERS = 3
DIM = 64
NUM_POS = BATCH * SEQ_LEN
VOCAB = 100000

_info = plsc.get_sparse_core_info()
_NC, _NS = _info.num_cores, _info.num_subcores
NW = _NC * _NS
POS_PER_W = NUM_POS // NW
CHUNK = 128
NCHUNK = POS_PER_W // CHUNK

_mesh = plsc.VectorSubcoreMesh(core_axis_name="c", subcore_axis_name="s")


HALF_VOCAB = VOCAB // 2


@functools.partial(
    pl.kernel,
    mesh=_mesh,
    out_type=jax.ShapeDtypeStruct((NUM_POS, NUM_LAYERS * DIM), jnp.float32),
    scratch_types=[
        pltpu.VMEM((NUM_LAYERS * POS_PER_W,), jnp.int32),
        pltpu.VMEM((NUM_LAYERS * POS_PER_W,), jnp.int32),
        pltpu.VMEM((CHUNK, 2 * DIM), jnp.float32),
        pltpu.VMEM((CHUNK, 2 * DIM), jnp.float32),
        pltpu.VMEM((CHUNK, NUM_LAYERS * DIM), jnp.float32),
        pltpu.VMEM((CHUNK, NUM_LAYERS * DIM), jnp.float32),
        pltpu.SemaphoreType.DMA,
        pltpu.SemaphoreType.DMA,
        pltpu.SemaphoreType.DMA,
        pltpu.SemaphoreType.DMA,
    ],
)
def _emb_lookup(
    tok_hbm, w6, w9, w12, out_hbm,
    tok_v, ixp_v, g0, g1, s0, s1, gsem0, gsem1, wsem0, wsem1,
):
    wid = lax.axis_index("c") * _NS + lax.axis_index("s")
    base = wid * POS_PER_W
    tables = (w6, w9, w12)
    gbufs = (g0, g1)
    gsems = (gsem0, gsem1)
    stages = (s0, s1)
    wsems = (wsem0, wsem1)

    for i in range(NUM_LAYERS):
        pltpu.sync_copy(
            tok_hbm.at[pl.ds(i * NUM_POS + base, POS_PER_W)],
            tok_v.at[pl.ds(i * POS_PER_W, POS_PER_W)],
        )

    def shift(j, cc):
        t = tok_v[pl.ds(j * 16, 16)]
        ixp_v[pl.ds(j * 16, 16)] = t >> 1
        return cc

    lax.fori_loop(0, NUM_LAYERS * POS_PER_W // 16, shift, 0)

    def issue_gather(i, c, buf, sem):
        off = i * POS_PER_W + c * CHUNK
        pltpu.async_copy(tables[i].at[ixp_v.at[pl.ds(off, CHUNK)]], buf, sem)

    def wait_gather(buf, sem):
        pltpu.make_async_copy(tables[0].at[pl.ds(0, CHUNK)], buf, sem).wait()

    def select(i, c, buf, st):
        def body(j, cc):
            tv = tok_v[pl.ds(i * POS_PER_W + c * CHUNK + j * 16, 16)]
            pv = (tv & 1) * DIM
            for lane in range(16):
                off = pv[lane]
                p = j * 16 + lane
                for kk in range(DIM // 16):
                    st[p, pl.ds(i * DIM + kk * 16, 16)] = buf[
                        p, pl.ds(off + kk * 16, 16)
                    ]
            return cc

        lax.fori_loop(0, CHUNK // 16, body, 0)

    issue_gather(0, 0, gbufs[0], gsems[0])

    def chunk_pair(t, carry):
        for half in range(2):
            c = 2 * t + half
            st = stages[half]

            @pl.when(t > 0)
            def _():
                pltpu.make_async_copy(
                    st, out_hbm.at[pl.ds(base, CHUNK)], wsems[half]
                ).wait()

            for i in range(NUM_LAYERS):
                kpar = (3 * half + i) % 2
                cur, csem = gbufs[kpar], gsems[kpar]
                nxt, nsem = gbufs[1 - kpar], gsems[1 - kpar]
                if i < 2:
                    issue_gather(i + 1, c, nxt, nsem)
                elif half == 0:
                    issue_gather(0, c + 1, nxt, nsem)
                else:

                    @pl.when(t < NCHUNK // 2 - 1)
                    def _():
                        issue_gather(0, c + 1, nxt, nsem)

                wait_gather(cur, csem)
                select(i, c, cur, st)

            pltpu.async_copy(
                st, out_hbm.at[pl.ds(base + c * CHUNK, CHUNK)], wsems[half]
            )
        return carry

    lax.fori_loop(0, NCHUNK // 2, chunk_pair, 0)

    for half in range(2):
        pltpu.make_async_copy(
            stages[half], out_hbm.at[pl.ds(base, CHUNK)], wsems[half]
        ).wait()


def kernel(tokens, W6, W9, W12):
    tok_t = jnp.transpose(
        tokens.reshape(NUM_POS, NUM_LAYERS).astype(jnp.int32)
    ).reshape(NUM_LAYERS * NUM_POS)
    wp6 = W6.reshape(HALF_VOCAB, 2 * DIM)
    wp9 = W9.reshape(HALF_VOCAB, 2 * DIM)
    wp12 = W12.reshape(HALF_VOCAB, 2 * DIM)
    out = _emb_lookup(tok_t, wp6, wp9, wp12)
    return out.reshape(BATCH, SEQ_LEN, NUM_LAYERS, DIM)


# speedup vs baseline: 1.4026x; 1.1237x over previous
import functools

import jax
import jax.numpy as jnp
from jax import lax
from jax.experimental import pallas as pl
from jax.experimental.pallas import tpu as pltpu
from jax.experimental.pallas import tpu_sc as plsc

BATCH = 16
SEQ_LEN = 2048
NUM_LAYERS = 3
DIM = 64
NUM_POS = BATCH * SEQ_LEN
VOCAB = 100000

_info = plsc.get_sparse_core_info()
_NC, _NS = _info.num_cores, _info.num_subcores
NW = _NC * _NS
POS_PER_W = NUM_POS // NW
CHUNK = 128
NCHUNK = POS_PER_W // CHUNK

_mesh = plsc.VectorSubcoreMesh(core_axis_name="c", subcore_axis_name="s")


HALF_VOCAB = VOCAB // 2


@functools.partial(
    pl.kernel,
    mesh=_mesh,
    out_type=jax.ShapeDtypeStruct((BATCH, SEQ_LEN, NUM_LAYERS * DIM), jnp.float32),
    scratch_types=[
        pltpu.VMEM((NUM_LAYERS * POS_PER_W,), jnp.int32),
        pltpu.VMEM((NUM_LAYERS * POS_PER_W,), jnp.int32),
        pltpu.VMEM((CHUNK, 2 * DIM), jnp.float32),
        pltpu.VMEM((CHUNK, 2 * DIM), jnp.float32),
        pltpu.VMEM((1, CHUNK, NUM_LAYERS * DIM), jnp.float32),
        pltpu.VMEM((1, CHUNK, NUM_LAYERS * DIM), jnp.float32),
        pltpu.SemaphoreType.DMA,
        pltpu.SemaphoreType.DMA,
        pltpu.SemaphoreType.DMA,
        pltpu.SemaphoreType.DMA,
    ],
)
def _emb_lookup(
    tok_hbm, w6, w9, w12, out_hbm,
    tok_v, ixp_v, g0, g1, s0, s1, gsem0, gsem1, wsem0, wsem1,
):
    wid = lax.axis_index("c") * _NS + lax.axis_index("s")
    base = wid * POS_PER_W
    bb = wid // 2
    tw = (wid % 2) * POS_PER_W
    tables = (w6, w9, w12)
    gbufs = (g0, g1)
    gsems = (gsem0, gsem1)
    stages = (s0, s1)
    wsems = (wsem0, wsem1)

    for i in range(NUM_LAYERS):
        pltpu.sync_copy(
            tok_hbm.at[pl.ds(i * NUM_POS + base, POS_PER_W)],
            tok_v.at[pl.ds(i * POS_PER_W, POS_PER_W)],
        )

    def shift(j, cc):
        t = tok_v[pl.ds(j * 16, 16)]
        ixp_v[pl.ds(j * 16, 16)] = t >> 1
        return cc

    lax.fori_loop(0, NUM_LAYERS * POS_PER_W // 16, shift, 0)

    def issue_gather(i, c, buf, sem):
        off = i * POS_PER_W + c * CHUNK
        pltpu.async_copy(tables[i].at[ixp_v.at[pl.ds(off, CHUNK)]], buf, sem)

    def wait_gather(buf, sem):
        pltpu.make_async_copy(tables[0].at[pl.ds(0, CHUNK)], buf, sem).wait()

    def select(i, c, buf, st):
        def body(j, cc):
            tv = tok_v[pl.ds(i * POS_PER_W + c * CHUNK + j * 16, 16)]
            pv = (tv & 1) * DIM
            for lane in range(16):
                off = pv[lane]
                p = j * 16 + lane
                for kk in range(DIM // 16):
                    st[0, p, pl.ds(i * DIM + kk * 16, 16)] = buf[
                        p, pl.ds(off + kk * 16, 16)
                    ]
            return cc

        lax.fori_loop(0, CHUNK // 16, body, 0)

    issue_gather(0, 0, gbufs[0], gsems[0])

    def chunk_pair(t, carry):
        for half in range(2):
            c = 2 * t + half
            st = stages[half]

            for i in range(NUM_LAYERS):
                kpar = (3 * half + i) % 2
                cur, csem = gbufs[kpar], gsems[kpar]
                nxt, nsem = gbufs[1 - kpar], gsems[1 - kpar]
                if i < 2:
                    issue_gather(i + 1, c, nxt, nsem)
                elif half == 0:
                    issue_gather(0, c + 1, nxt, nsem)
                else:

                    @pl.when(t < NCHUNK // 2 - 1)
                    def _():
                        issue_gather(0, c + 1, nxt, nsem)

                wait_gather(cur, csem)
                select(i, c, cur, st)

            pltpu.sync_copy(
                st, out_hbm.at[pl.ds(bb, 1), pl.ds(tw + c * CHUNK, CHUNK)]
            )
        return carry

    lax.fori_loop(0, NCHUNK // 2, chunk_pair, 0)


def kernel(tokens, W6, W9, W12):
    tok_t = jnp.transpose(
        tokens.reshape(NUM_POS, NUM_LAYERS).astype(jnp.int32)
    ).reshape(NUM_LAYERS * NUM_POS)
    wp6 = W6.reshape(HALF_VOCAB, 2 * DIM)
    wp9 = W9.reshape(HALF_VOCAB, 2 * DIM)
    wp12 = W12.reshape(HALF_VOCAB, 2 * DIM)
    out = _emb_lookup(tok_t, wp6, wp9, wp12)
    return out.reshape(BATCH, SEQ_LEN, NUM_LAYERS, DIM)
